# baseline (speedup 1.0000x reference)
import functools

import jax
import jax.numpy as jnp
from jax import lax
from jax.experimental import pallas as pl
from jax.experimental.pallas import tpu as pltpu

B, S, LANES = 4, 8192, 2048
SB = 64
LB = 2048
NS = S // SB


def _tc_body(x_ref, o_ref, carry_ref):
    s = pl.program_id(1)

    @pl.when(s == 0)
    def _reset():
        carry_ref[...] = jnp.zeros_like(carry_ref)

    x = x_ref[0]
    i = lax.broadcasted_iota(jnp.int32, (SB, SB), 0)
    j = lax.broadcasted_iota(jnp.int32, (SB, SB), 1)
    tri = (i >= j).astype(jnp.float32)
    c = lax.dot(tri, x, precision=lax.Precision.HIGHEST,
                preferred_element_type=jnp.float32)
    c = c + carry_ref[0:1, :]
    o_ref[0] = c
    carry_ref[...] = c[SB - 1:SB, :] * jnp.ones((8, 1), jnp.float32)


def _cumsum_tc(x):
    return pl.pallas_call(
        _tc_body,
        grid=(B, NS),
        in_specs=[pl.BlockSpec((1, SB, LB), lambda b, s: (b, s, 0))],
        out_specs=pl.BlockSpec((1, SB, LB), lambda b, s: (b, s, 0)),
        out_shape=jax.ShapeDtypeStruct((B, S, LANES), jnp.float32),
        scratch_shapes=[pltpu.VMEM((8, LB), jnp.float32)],
        compiler_params=pltpu.CompilerParams(
            dimension_semantics=("parallel", "arbitrary"),
        ),
    )(x)


def kernel(masks):
    return _cumsum_tc(masks)
